# baseline (speedup 1.0000x reference)
import functools

import jax
import jax.numpy as jnp
from jax import lax
from jax.experimental import pallas as pl
from jax.experimental.pallas import tpu as pltpu
from jax.experimental.pallas import tpu_sc as plsc

N = 65536
C = 20
M = 256
J = 16
D = C * J

NC = 2
NS = 16
NW = NC * NS
ROWS_PER_W = N // NW
CHUNK = 128
NCHUNK = ROWS_PER_W // CHUNK
NBUF = 2


def _table_body(lb_ref, lpi_ref, post_ref, ll_ref):
    lam = lb_ref[:]
    mx = jnp.max(lam, axis=1, keepdims=True)
    e = jnp.exp(lam - mx)
    B = e / jnp.sum(e, axis=1, keepdims=True)
    lpi = lpi_ref[:]
    pmx = jnp.max(lpi, axis=0, keepdims=True)
    pe = jnp.exp(lpi - pmx)
    Pi = pe / jnp.sum(pe, axis=0, keepdims=True)
    T = B * Pi[:, None, :]
    denom = jnp.sum(T, axis=0)
    post_ref[:] = T / denom[None, :, :]
    ll_ref[:] = jnp.log(denom)


_table = pl.pallas_call(
    _table_body,
    out_shape=(
        jax.ShapeDtypeStruct((C, M, J), jnp.float32),
        jax.ShapeDtypeStruct((M, J), jnp.float32),
    ),
)


def _gather_body(ptab_hbm, lltab_hbm, idx_hbm, outp_hbm, outl_hbm,
                 idx_v, rows_v, llrows_v, semp, seml):
    wid = lax.axis_index("s") * NC + lax.axis_index("c")
    base = wid * ROWS_PER_W
    pltpu.sync_copy(idx_hbm.at[pl.ds(wid * NCHUNK, NCHUNK)], idx_v)

    gathers = [None] * NCHUNK

    def start(c):
        b = c % NBUF
        gp = pltpu.async_copy(ptab_hbm.at[idx_v.at[c]], rows_v.at[b], semp)
        gl = pltpu.async_copy(lltab_hbm.at[idx_v.at[c]], llrows_v.at[b], seml)
        gathers[c] = (gp, gl)

    start(0)
    for c in range(NCHUNK):
        if c + 1 < NCHUNK:
            start(c + 1)
        b = c % NBUF
        gp, gl = gathers[c]
        gp.wait()
        gl.wait()
        row0 = base + c * CHUNK
        pltpu.sync_copy(rows_v.at[b], outp_hbm.at[pl.ds(row0, CHUNK)])
        pltpu.sync_copy(llrows_v.at[b], outl_hbm.at[pl.ds(row0, CHUNK)])


@functools.cache
def _gather():
    return pl.kernel(
        _gather_body,
        mesh=plsc.VectorSubcoreMesh(
            core_axis_name="c", subcore_axis_name="s",
            num_cores=NC, num_subcores=NS),
        out_type=[
            jax.ShapeDtypeStruct((N, C, J), jnp.float32),
            jax.ShapeDtypeStruct((N, J), jnp.float32),
        ],
        scratch_types=[
            pltpu.VMEM((NCHUNK, CHUNK), jnp.int32),
            pltpu.VMEM((NBUF, CHUNK, C, J), jnp.float32),
            pltpu.VMEM((NBUF, CHUNK, J), jnp.float32),
            pltpu.SemaphoreType.DMA,
            pltpu.SemaphoreType.DMA,
        ],
        compiler_params=pltpu.CompilerParams(use_tc_tiling_on_sc=False),
    )


def kernel(x, lambda_B, lambda_Pi):
    post_t, ll_t = _table(lambda_B, lambda_Pi)
    ptab = jnp.transpose(post_t, (1, 0, 2))
    idx = x.astype(jnp.int32).reshape(N // CHUNK, CHUNK)
    outp, outl = _gather()(ptab, ll_t, idx)
    return (outl, outp)


# speedup vs baseline: 3.3418x; 3.3418x over previous
import functools

import jax
import jax.numpy as jnp
from jax import lax
from jax.experimental import pallas as pl
from jax.experimental.pallas import tpu as pltpu
from jax.experimental.pallas import tpu_sc as plsc

N = 65536
C = 20
M = 256
J = 16
KP = C * J

NC = 2
NS = 16
NW = NC * NS
NPW = N // NW
L = 16
NG = NPW // L


def _table_body(lbt_ref, lpi_ref, post_ref, ll_ref):
    lam = lbt_ref[:]
    mx = jnp.max(lam, axis=2, keepdims=True)
    e = jnp.exp(lam - mx)
    B = e / jnp.sum(e, axis=2, keepdims=True)
    lpi = lpi_ref[:]
    pmx = jnp.max(lpi, axis=0, keepdims=True)
    pe = jnp.exp(lpi - pmx)
    Pi = pe / jnp.sum(pe, axis=0, keepdims=True)
    T = B * Pi[:, :, None]
    denom = jnp.sum(T, axis=0)
    post_ref[:] = T / denom[None, :, :]
    ll_ref[:] = jnp.log(denom)


_table = pl.pallas_call(
    _table_body,
    out_shape=(
        jax.ShapeDtypeStruct((C, J, M), jnp.float32),
        jax.ShapeDtypeStruct((J, M), jnp.float32),
    ),
)


def _expand_body(tabp_hbm, tabl_hbm, x_hbm, outp_hbm, outl_hbm,
                 tabp_v, tabl_v, x_v, obuf):
    wid = lax.axis_index("s") * NC + lax.axis_index("c")
    nbase = wid * NPW
    pltpu.sync_copy(tabp_hbm, tabp_v)
    pltpu.sync_copy(tabl_hbm, tabl_v)
    pltpu.sync_copy(x_hbm.at[pl.ds(nbase, NPW)], x_v)

    def fill(tab_v, kbase):
        def inner(g4, _):
            for u in range(4):
                off = (g4 * 4 + u) * L
                xv = x_v[pl.ds(off, L)]
                for jj in range(8):
                    idx = xv + (kbase + jj) * M
                    obuf[jj, pl.ds(off, L)] = plsc.load_gather(tab_v, [idx])
            return 0
        lax.fori_loop(0, NG // 4, inner, 0, unroll=False)

    def pgroup(grp, _):
        c = grp // 2
        j8 = grp % 2
        fill(tabp_v, grp * 8)
        pltpu.sync_copy(
            obuf, outp_hbm.at[c, pl.ds(j8 * 8, 8), pl.ds(nbase, NPW)])
        return 0

    lax.fori_loop(0, KP // 8, pgroup, 0, unroll=False)

    for j8 in range(2):
        fill(tabl_v, j8 * 8)
        pltpu.sync_copy(
            obuf, outl_hbm.at[pl.ds(j8 * 8, 8), pl.ds(nbase, NPW)])


@functools.cache
def _expand():
    return pl.kernel(
        _expand_body,
        mesh=plsc.VectorSubcoreMesh(
            core_axis_name="c", subcore_axis_name="s",
            num_cores=NC, num_subcores=NS),
        out_type=[
            jax.ShapeDtypeStruct((C, J, N), jnp.float32),
            jax.ShapeDtypeStruct((J, N), jnp.float32),
        ],
        scratch_types=[
            pltpu.VMEM((KP * M,), jnp.float32),
            pltpu.VMEM((J * M,), jnp.float32),
            pltpu.VMEM((NPW,), jnp.int32),
            pltpu.VMEM((8, NPW), jnp.float32),
        ],
        compiler_params=pltpu.CompilerParams(
            use_tc_tiling_on_sc=True, needs_layout_passes=False),
    )


def kernel(x, lambda_B, lambda_Pi):
    lbt = jnp.transpose(lambda_B, (0, 2, 1))
    post_t, ll_t = _table(lbt, lambda_Pi)
    tabp = post_t.reshape(KP * M)
    tabl = ll_t.reshape(J * M)
    xi = x.astype(jnp.int32)
    outp, outl = _expand()(tabp, tabl, xi)
    return (jnp.transpose(outl, (1, 0)), jnp.transpose(outp, (2, 0, 1)))


# speedup vs baseline: 9.1278x; 2.7314x over previous
import functools

import jax
import jax.numpy as jnp
from jax import lax
from jax.experimental import pallas as pl
from jax.experimental.pallas import tpu as pltpu
from jax.experimental.pallas import tpu_sc as plsc

N = 65536
C = 20
M = 256
J = 16
KP = C * J

NC = 2
NS = 16
NW = NC * NS
NPW = N // NW
L = 16
NG = NPW // L


def _table_body(lbt_ref, lpi_ref, post_ref, ll_ref):
    lam = lbt_ref[:]
    mx = jnp.max(lam, axis=2, keepdims=True)
    e = jnp.exp(lam - mx)
    B = e / jnp.sum(e, axis=2, keepdims=True)
    lpi = lpi_ref[:]
    pmx = jnp.max(lpi, axis=0, keepdims=True)
    pe = jnp.exp(lpi - pmx)
    Pi = pe / jnp.sum(pe, axis=0, keepdims=True)
    T = B * Pi[:, :, None]
    denom = jnp.sum(T, axis=0)
    post_ref[:] = T / denom[None, :, :]
    ll_ref[:] = jnp.log(denom)


_table = pl.pallas_call(
    _table_body,
    out_shape=(
        jax.ShapeDtypeStruct((C, J, M), jnp.float32),
        jax.ShapeDtypeStruct((J, M), jnp.float32),
    ),
)


def _expand_body(tabp_hbm, tabl_hbm, x_hbm, outp_hbm, outl_hbm,
                 tabp_v, tabl_v, x_v, obuf):
    wid = lax.axis_index("s") * NC + lax.axis_index("c")
    nbase = wid * NPW
    pltpu.sync_copy(tabp_hbm, tabp_v)
    pltpu.sync_copy(tabl_hbm, tabl_v)
    pltpu.sync_copy(x_hbm.at[pl.ds(nbase, NPW)], x_v)

    def fill(tab_v, kbase):
        tab8 = tab_v.at[pl.ds(kbase * M, 8 * M)]

        @plsc.parallel_loop(0, NG, 1, unroll=4)
        def inner(g):
            off = g * L
            xv = x_v[pl.ds(off, L)]
            for jj in range(8):
                obuf[jj, pl.ds(off, L)] = plsc.load_gather(tab8, [xv + jj * M])

    def pgroup(grp, _):
        c = grp // 2
        j8 = grp % 2
        fill(tabp_v, grp * 8)
        pltpu.sync_copy(
            obuf, outp_hbm.at[c, pl.ds(j8 * 8, 8), pl.ds(nbase, NPW)])
        return 0

    lax.fori_loop(0, KP // 8, pgroup, 0, unroll=False)

    for j8 in range(2):
        fill(tabl_v, j8 * 8)
        pltpu.sync_copy(
            obuf, outl_hbm.at[pl.ds(j8 * 8, 8), pl.ds(nbase, NPW)])


@functools.cache
def _expand():
    return pl.kernel(
        _expand_body,
        mesh=plsc.VectorSubcoreMesh(
            core_axis_name="c", subcore_axis_name="s",
            num_cores=NC, num_subcores=NS),
        out_type=[
            jax.ShapeDtypeStruct((C, J, N), jnp.float32),
            jax.ShapeDtypeStruct((J, N), jnp.float32),
        ],
        scratch_types=[
            pltpu.VMEM((KP * M,), jnp.float32),
            pltpu.VMEM((J * M,), jnp.float32),
            pltpu.VMEM((NPW,), jnp.int32),
            pltpu.VMEM((8, NPW), jnp.float32),
        ],
        compiler_params=pltpu.CompilerParams(
            use_tc_tiling_on_sc=True, needs_layout_passes=False),
    )


def kernel(x, lambda_B, lambda_Pi):
    lbt = jnp.transpose(lambda_B, (0, 2, 1))
    post_t, ll_t = _table(lbt, lambda_Pi)
    tabp = post_t.reshape(KP * M)
    tabl = ll_t.reshape(J * M)
    xi = x.astype(jnp.int32)
    outp, outl = _expand()(tabp, tabl, xi)
    return (jnp.transpose(outl, (1, 0)), jnp.transpose(outp, (2, 0, 1)))


# speedup vs baseline: 11.3457x; 1.2430x over previous
import functools

import jax
import jax.numpy as jnp
from jax import lax
from jax.experimental import pallas as pl
from jax.experimental.pallas import tpu as pltpu
from jax.experimental.pallas import tpu_sc as plsc

N = 65536
C = 20
M = 256
J = 16
KP = C * J

NC = 2
NS = 16
NW = NC * NS
NPW = N // NW
L = 16
NG = NPW // L


def _table_body(lbt_ref, lpi_ref, post_ref, ll_ref):
    lam = lbt_ref[:]
    mx = jnp.max(lam, axis=2, keepdims=True)
    e = jnp.exp(lam - mx)
    B = e / jnp.sum(e, axis=2, keepdims=True)
    lpi = lpi_ref[:]
    pmx = jnp.max(lpi, axis=0, keepdims=True)
    pe = jnp.exp(lpi - pmx)
    Pi = pe / jnp.sum(pe, axis=0, keepdims=True)
    T = B * Pi[:, :, None]
    denom = jnp.sum(T, axis=0)
    post_ref[:] = T / denom[None, :, :]
    ll_ref[:] = jnp.log(denom)


_table = pl.pallas_call(
    _table_body,
    out_shape=(
        jax.ShapeDtypeStruct((C, J, M), jnp.float32),
        jax.ShapeDtypeStruct((J, M), jnp.float32),
    ),
)


def _expand_body(tabp_hbm, tabl_hbm, x_hbm, outp_hbm, outl_hbm,
                 tabp_v, tabl_v, x_v, obuf, sems):
    wid = lax.axis_index("s") * NC + lax.axis_index("c")
    nbase = wid * NPW
    pltpu.sync_copy(tabp_hbm, tabp_v)
    pltpu.sync_copy(tabl_hbm, tabl_v)
    pltpu.sync_copy(x_hbm.at[pl.ds(nbase, NPW)], x_v)

    def fill(tab_v, kbase, b):
        tab8 = tab_v.at[pl.ds(kbase * M, 8 * M)]
        ob = obuf.at[b]

        @plsc.parallel_loop(0, NG, 1, unroll=4)
        def inner(g):
            off = g * L
            xv = x_v[pl.ds(off, L)]
            for jj in range(8):
                ob[jj, pl.ds(off, L)] = plsc.load_gather(tab8, [xv + jj * M])

    groups = [(tabp_v, grp * 8,
               outp_hbm.at[grp // 2, pl.ds((grp % 2) * 8, 8),
                           pl.ds(nbase, NPW)])
              for grp in range(KP // 8)]
    groups += [(tabl_v, j8 * 8,
                outl_hbm.at[pl.ds(j8 * 8, 8), pl.ds(nbase, NPW)])
               for j8 in range(2)]

    pending = [None, None]
    for i, (tv, kb, dst) in enumerate(groups):
        b = i % 2
        if pending[b] is not None:
            pending[b].wait()
        fill(tv, kb, b)
        pending[b] = pltpu.async_copy(obuf.at[b], dst, sems[b])
    for p in pending:
        p.wait()


@functools.cache
def _expand():
    return pl.kernel(
        _expand_body,
        mesh=plsc.VectorSubcoreMesh(
            core_axis_name="c", subcore_axis_name="s",
            num_cores=NC, num_subcores=NS),
        out_type=[
            jax.ShapeDtypeStruct((C, J, N), jnp.float32),
            jax.ShapeDtypeStruct((J, N), jnp.float32),
        ],
        scratch_types=[
            pltpu.VMEM((KP * M,), jnp.float32),
            pltpu.VMEM((J * M,), jnp.float32),
            pltpu.VMEM((NPW,), jnp.int32),
            pltpu.VMEM((2, 8, NPW), jnp.float32),
            [pltpu.SemaphoreType.DMA, pltpu.SemaphoreType.DMA],
        ],
        compiler_params=pltpu.CompilerParams(
            use_tc_tiling_on_sc=True, needs_layout_passes=False),
    )


def kernel(x, lambda_B, lambda_Pi):
    lbt = jnp.transpose(lambda_B, (0, 2, 1))
    post_t, ll_t = _table(lbt, lambda_Pi)
    tabp = post_t.reshape(KP * M)
    tabl = ll_t.reshape(J * M)
    xi = x.astype(jnp.int32)
    outp, outl = _expand()(tabp, tabl, xi)
    return (jnp.transpose(outl, (1, 0)), jnp.transpose(outp, (2, 0, 1)))
